# speedup vs baseline: 1.7734x; 1.7734x over previous
from math import sqrt

import functools

import jax
import jax.numpy as jnp
from jax import lax
from jax.experimental import pallas as pl
from jax.experimental.pallas import tpu as pltpu

_MASK_VALUE = -1e30


def _fused_attn_kernel(x_ref, wqkv_ref, bqkv_ref, wo_ref, bo_ref,
                       y_ref, a_ref, qkv_scr, *, n_heads, d_keys, lt, scale):
    i = pl.program_id(1)
    H, dk = n_heads, d_keys
    hd = H * dk
    L = x_ref.shape[1]

    @pl.when(i == 0)
    def _project():
        qkv_scr[...] = (
            jnp.dot(x_ref[0], wqkv_ref[...],
                    preferred_element_type=jnp.float32)
            + bqkv_ref[...]
        )

    q_all = qkv_scr[pl.ds(i * lt, lt), 0:hd] * scale

    row = i * lt + lax.broadcasted_iota(jnp.int32, (lt, L), 0)
    col = lax.broadcasted_iota(jnp.int32, (lt, L), 1)
    causal = col > row

    outs = []
    for h in range(H):
        q = q_all[:, h * dk:(h + 1) * dk]
        k = qkv_scr[:, hd + h * dk: hd + (h + 1) * dk]
        v = qkv_scr[:, 2 * hd + h * dk: 2 * hd + (h + 1) * dk]
        s = lax.dot_general(q, k, (((1,), (1,)), ((), ())),
                            preferred_element_type=jnp.float32)
        s = jnp.where(causal, _MASK_VALUE, s)
        m = jnp.max(s, axis=-1, keepdims=True)
        p = jnp.exp(s - m)
        denom = jnp.sum(p, axis=-1, keepdims=True)
        a = p * (1.0 / denom)
        a_ref[0, h] = a.astype(a_ref.dtype)
        outs.append(lax.dot_general(a, v, (((1,), (0,)), ((), ())),
                                    preferred_element_type=jnp.float32))
    acc = jnp.concatenate(outs, axis=1)
    y_ref[0] = (jnp.dot(acc, wo_ref[...],
                        preferred_element_type=jnp.float32)
                + bo_ref[...]).astype(y_ref.dtype)


def kernel(x, wqkv3, bqkv3, wo3, bo):
    B, L, d_model = x.shape
    G, _, dk = wqkv3.shape
    H = G // 3
    hd = H * dk
    lt = 128 if L % 128 == 0 else L
    scale = 1.0 / sqrt(dk)

    wqkv_flat = jnp.transpose(wqkv3, (1, 0, 2)).reshape(d_model, G * dk)
    bqkv_flat = bqkv3.reshape(1, G * dk)
    wo_flat = wo3.reshape(hd, d_model)

    kern = functools.partial(_fused_attn_kernel, n_heads=H, d_keys=dk,
                             lt=lt, scale=scale)
    y, attn = pl.pallas_call(
        kern,
        out_shape=(
            jax.ShapeDtypeStruct((B, L, d_model), x.dtype),
            jax.ShapeDtypeStruct((B, H, L, L), x.dtype),
        ),
        grid_spec=pltpu.PrefetchScalarGridSpec(
            num_scalar_prefetch=0,
            grid=(B, L // lt),
            in_specs=[
                pl.BlockSpec((1, L, d_model), lambda b, i: (b, 0, 0)),
                pl.BlockSpec((d_model, G * dk), lambda b, i: (0, 0)),
                pl.BlockSpec((1, G * dk), lambda b, i: (0, 0)),
                pl.BlockSpec((hd, d_model), lambda b, i: (0, 0)),
                pl.BlockSpec((1, d_model), lambda b, i: (0, 0)),
            ],
            out_specs=(
                pl.BlockSpec((1, lt, d_model), lambda b, i: (b, i, 0)),
                pl.BlockSpec((1, H, lt, L), lambda b, i: (b, 0, i, 0)),
            ),
            scratch_shapes=[pltpu.VMEM((L, G * dk), jnp.float32)],
        ),
        compiler_params=pltpu.CompilerParams(
            dimension_semantics=("parallel", "arbitrary"),
            vmem_limit_bytes=60 * 1024 * 1024,
        ),
    )(x, wqkv_flat, bqkv_flat, wo_flat, bo)
    return y, attn
